# baseline (speedup 1.0000x reference)
import functools

import numpy as np
import jax
import jax.numpy as jnp
from jax import lax
from jax.experimental import pallas as pl
from jax.experimental.pallas import tpu as pltpu
from jax.experimental.pallas import tpu_sc as plsc

_FEATURES_SHAPE = (50, 50)
_STRIDE = 16
_ANCHOR_SIZE = 16
_NUM_CLASSES = 20
_NUM_GT = 100
_GT_PAD = 112
_NEG_OVERLAP = 0.4
_POS_OVERLAP = 0.5

_NC, _NS, _L = 2, 16, 16
_NW = _NC * _NS
_TOTAL = _FEATURES_SHAPE[0] * _FEATURES_SHAPE[1] * 9
_PAD_TOTAL = 22528
_APW = _PAD_TOTAL // _NW
_STEPS = _APW // _L
_BLK = 2
_GT_UNROLL = 2

_LN2 = 0.6931471805599453
_SQRT2 = 1.4142135623730951


def _generate_base_anchors(base_size=16):
    ratios = np.array([0.5, 1.0, 2.0])
    scales = np.array([8.0, 16.0, 32.0])
    w = float(base_size); h = float(base_size)
    x_ctr = 0.5 * (w - 1.0); y_ctr = 0.5 * (h - 1.0)
    size = w * h
    size_ratios = size / ratios
    ws = np.round(np.sqrt(size_ratios))
    hs = np.round(ws * ratios)
    ws = (ws[:, None] * scales[None, :]).reshape(-1)
    hs = (hs[:, None] * scales[None, :]).reshape(-1)
    anchors = np.stack([x_ctr - 0.5 * (ws - 1.0),
                        y_ctr - 0.5 * (hs - 1.0),
                        x_ctr + 0.5 * (ws - 1.0),
                        y_ctr + 0.5 * (hs - 1.0)], axis=1)
    return anchors.astype(np.float32)


def _shift_anchors(shape, stride, anchors):
    shift_x = np.arange(0, shape[1]) * stride
    shift_y = np.arange(0, shape[0]) * stride
    sx, sy = np.meshgrid(shift_x, shift_y)
    shifts = np.stack([sx.ravel(), sy.ravel(), sx.ravel(), sy.ravel()],
                      axis=1).astype(np.float32)
    return (anchors[None, :, :] + shifts[:, None, :]).reshape(-1, 4)


_ANCHORS_NP = _shift_anchors(_FEATURES_SHAPE, _STRIDE,
                             _generate_base_anchors(_ANCHOR_SIZE))


def _anchor_consts():
    a = np.concatenate(
        [_ANCHORS_NP, np.broadcast_to(_ANCHORS_NP[:1], (_PAD_TOTAL - _TOTAL, 4))],
        axis=0).astype(np.float64)
    x1, y1, x2, y2 = a[:, 0], a[:, 1], a[:, 2], a[:, 3]
    w = x2 - x1 + 1.0
    h = y2 - y1 + 1.0
    rows = np.stack([
        x1, y1, x2, y2,
        w * h,
        x1 + 0.5 * w,
        y1 + 0.5 * h,
        1.0 / w,
        1.0 / h,
        np.log(w),
        np.log(h),
        ((x1 >= 0.0) & (y1 >= 0.0)).astype(np.float64),
    ], axis=0).astype(np.float32)
    return np.ascontiguousarray(
        rows.reshape(12, _NW, _APW).transpose(1, 0, 2))


_ACONST_NP = _anchor_consts()

_AX1, _AY1, _AX2, _AY2, _AAREA, _ACX, _ACY, _AINVW, _AINVH, _ALOGW, _ALOGH, _AIB = range(12)


def _softlog(x):
    b = lax.bitcast_convert_type(x, jnp.int32)
    e = ((b >> 23) & 0xFF) - 127
    m = lax.bitcast_convert_type((b & 0x007FFFFF) | 0x3F800000, jnp.float32)
    big = m > _SQRT2
    m = jnp.where(big, m * 0.5, m)
    e = jnp.where(big, e + 1, e)
    s = (m - 1.0) / (m + 1.0)
    t = s * s
    p = 2.0 * s * (1.0 + t * (1.0 / 3.0 + t * (1.0 / 5.0 + t * (1.0 / 7.0))))
    return e.astype(jnp.float32) * _LN2 + p


def _sc_body(acst_hbm, graw_hbm, im_hbm,
             lab_hbm, bb_hbm, ct_hbm,
             acst_v, graw_v, im_v2,
             gx1_v, gy1_v, gx2_v, gy2_v, garea_v,
             gcx_v, gcy_v, glw_v, glh_v, gcls_v,
             lab_v, bb_v, ct_v):
    wid = lax.axis_index("s") * _NC + lax.axis_index("c")
    base = wid * _APW

    pltpu.sync_copy(acst_hbm.at[wid], acst_v)
    pltpu.sync_copy(graw_hbm, graw_v)
    pltpu.sync_copy(im_hbm, im_v2)

    zeros16 = jnp.zeros((_L,), jnp.float32)
    ones16 = jnp.ones((_L,), jnp.float32)
    lane = lax.iota(jnp.int32, _L)

    h_img = im_v2[0, :]
    w_img = im_v2[1, :]

    for t in range(_GT_PAD // _L):
        s = pl.ds(t * _L, _L)
        x1 = graw_v[0, s]; y1 = graw_v[1, s]
        x2 = graw_v[2, s]; y2 = graw_v[3, s]
        c = graw_v[4, s]
        gw = x2 - x1 + 1.0
        gh = y2 - y1 + 1.0
        gx1_v[s] = x1; gy1_v[s] = y1
        gx2_v[s] = x2; gy2_v[s] = y2
        garea_v[s] = gw * gh
        gcx_v[s] = x1 + 0.5 * gw
        gcy_v[s] = y1 + 0.5 * gh
        glw_v[s] = _softlog(gw)
        glh_v[s] = _softlog(gh)
        gcls_v[s] = c

    def blk_fn(blk, _):
        ax1 = []; ay1 = []; ax2 = []; ay2 = []; aarea = []
        for t in range(_BLK):
            s = pl.ds((blk * _BLK + t) * _L, _L)
            ax1.append(acst_v[_AX1, s]); ay1.append(acst_v[_AY1, s])
            ax2.append(acst_v[_AX2, s]); ay2.append(acst_v[_AY2, s])
            aarea.append(acst_v[_AAREA, s])

        def gt_fn(jj, carry):
            bi = list(carry[0:_BLK])
            bu = list(carry[_BLK:2 * _BLK])
            bx = list(carry[2 * _BLK:3 * _BLK])
            for u in range(_GT_UNROLL):
                j = jj * _GT_UNROLL + u
                jb = jnp.full((_L,), j, jnp.int32)
                gx1 = plsc.load_gather(gx1_v, [jb])
                gy1 = plsc.load_gather(gy1_v, [jb])
                gx2 = plsc.load_gather(gx2_v, [jb])
                gy2 = plsc.load_gather(gy2_v, [jb])
                garea = plsc.load_gather(garea_v, [jb])
                for t in range(_BLK):
                    iw = jnp.minimum(ax2[t], gx2) - jnp.maximum(ax1[t], gx1) + 1.0
                    ih = jnp.minimum(ay2[t], gy2) - jnp.maximum(ay1[t], gy1) + 1.0
                    iw = jnp.maximum(iw, 0.0)
                    ih = jnp.maximum(ih, 0.0)
                    inter = iw * ih
                    union = aarea[t] + garea - inter
                    m = inter * bu[t] > bi[t] * union
                    bi[t] = jnp.where(m, inter, bi[t])
                    bu[t] = jnp.where(m, union, bu[t])
                    bx[t] = jnp.where(m, jb, bx[t])
            return tuple(bi) + tuple(bu) + tuple(bx)

        init = (tuple(jnp.full((_L,), -1.0, jnp.float32) for _ in range(_BLK))
                + tuple(jnp.ones((_L,), jnp.float32) for _ in range(_BLK))
                + tuple(jnp.zeros((_L,), jnp.int32) for _ in range(_BLK)))
        carry = lax.fori_loop(0, _NUM_GT // _GT_UNROLL, gt_fn, init)

        for t in range(_BLK):
            step = blk * _BLK + t
            s = pl.ds(step * _L, _L)
            best = carry[t] / carry[_BLK + t]
            bidx = carry[2 * _BLK + t]

            gcx = plsc.load_gather(gcx_v, [bidx])
            gcy = plsc.load_gather(gcy_v, [bidx])
            glw = plsc.load_gather(glw_v, [bidx])
            glh = plsc.load_gather(glh_v, [bidx])
            gcls = plsc.load_gather(gcls_v, [bidx])

            lab = jnp.full((_L,), -2.0, jnp.float32)
            lab = jnp.where(best < _NEG_OVERLAP, -1.0, lab)
            lab = jnp.where(best >= _POS_OVERLAP, 1.0, lab)

            inside = ((acst_v[_AIB, s] > 0.0)
                      & (ax2[t] < w_img) & (ay2[t] < h_img))
            lab = jnp.where(inside, lab, -2.0)
            lab = jnp.where(lab == 1.0, gcls, lab)

            dx = (gcx - acst_v[_ACX, s]) * acst_v[_AINVW, s]
            dy = (gcy - acst_v[_ACY, s]) * acst_v[_AINVH, s]
            dw = glw - acst_v[_ALOGW, s]
            dh = glh - acst_v[_ALOGH, s]

            lab_v[s] = lab
            arow = step * _L + lane
            bb_idx = arow * 4
            plsc.store_scatter(bb_v, [bb_idx], dx)
            plsc.store_scatter(bb_v, [bb_idx + 1], dy)
            plsc.store_scatter(bb_v, [bb_idx + 2], dw)
            plsc.store_scatter(bb_v, [bb_idx + 3], dh)

            for k in range(_NUM_CLASSES):
                ct_v[pl.ds(step * (_L * _NUM_CLASSES) + k * _L, _L)] = zeros16
            cls_i = lab.astype(jnp.int32)
            valid = (cls_i >= 0) & (cls_i < _NUM_CLASSES)
            plsc.store_scatter(ct_v, [arow * _NUM_CLASSES + cls_i], ones16,
                               mask=valid)
        return 0

    lax.fori_loop(0, _STEPS // _BLK, blk_fn, 0)

    pltpu.sync_copy(lab_v, lab_hbm.at[pl.ds(base, _APW)])
    pltpu.sync_copy(bb_v, bb_hbm.at[pl.ds(base * 4, _APW * 4)])
    pltpu.sync_copy(ct_v, ct_hbm.at[pl.ds(base * _NUM_CLASSES,
                                          _APW * _NUM_CLASSES)])


@functools.partial(jax.jit, static_argnames=())
def _run_sc(acst, graw, im_pad):
    mesh = plsc.VectorSubcoreMesh(core_axis_name="c", subcore_axis_name="s",
                                  num_cores=_NC, num_subcores=_NS)
    f = pl.kernel(
        _sc_body,
        out_type=(
            jax.ShapeDtypeStruct((_PAD_TOTAL,), jnp.float32),
            jax.ShapeDtypeStruct((_PAD_TOTAL * 4,), jnp.float32),
            jax.ShapeDtypeStruct((_PAD_TOTAL * _NUM_CLASSES,), jnp.float32),
        ),
        mesh=mesh,
        compiler_params=pltpu.CompilerParams(needs_layout_passes=False),
        scratch_types=[
            pltpu.VMEM((12, _APW), jnp.float32),
            pltpu.VMEM((5, _GT_PAD), jnp.float32),
            pltpu.VMEM((2, _L), jnp.float32),
            pltpu.VMEM((_GT_PAD,), jnp.float32),
            pltpu.VMEM((_GT_PAD,), jnp.float32),
            pltpu.VMEM((_GT_PAD,), jnp.float32),
            pltpu.VMEM((_GT_PAD,), jnp.float32),
            pltpu.VMEM((_GT_PAD,), jnp.float32),
            pltpu.VMEM((_GT_PAD,), jnp.float32),
            pltpu.VMEM((_GT_PAD,), jnp.float32),
            pltpu.VMEM((_GT_PAD,), jnp.float32),
            pltpu.VMEM((_GT_PAD,), jnp.float32),
            pltpu.VMEM((_GT_PAD,), jnp.float32),
            pltpu.VMEM((_APW,), jnp.float32),
            pltpu.VMEM((_APW * 4,), jnp.float32),
            pltpu.VMEM((_APW * _NUM_CLASSES,), jnp.float32),
        ],
    )
    return f(acst, graw, im_pad)


def kernel(im_info, gt_boxes):
    gt = gt_boxes[0]
    graw = jnp.concatenate(
        [gt.T, jnp.zeros((5, _GT_PAD - _NUM_GT), jnp.float32)], axis=1)
    im_pad = jnp.broadcast_to(im_info[0, :2, None], (2, _L))
    acst = jnp.asarray(_ACONST_NP)

    lab, bb, ct = _run_sc(acst, graw, im_pad)

    labels = lab[:_TOTAL][None]
    class_targets = ct.reshape(_PAD_TOTAL, _NUM_CLASSES)[:_TOTAL][None]
    bbox_targets = bb.reshape(_PAD_TOTAL, 4)[:_TOTAL][None]
    anchors = jnp.asarray(_ANCHORS_NP)[None]
    return (labels, class_targets, bbox_targets, anchors)


# speedup vs baseline: 1.7948x; 1.7948x over previous
import functools

import numpy as np
import jax
import jax.numpy as jnp
from jax import lax
from jax.experimental import pallas as pl
from jax.experimental.pallas import tpu as pltpu
from jax.experimental.pallas import tpu_sc as plsc

_FEATURES_SHAPE = (50, 50)
_STRIDE = 16
_ANCHOR_SIZE = 16
_NUM_CLASSES = 20
_NUM_GT = 100
_GT_PAD = 112
_NEG_OVERLAP = 0.4
_POS_OVERLAP = 0.5

_NC, _NS, _L = 2, 16, 16
_NW = _NC * _NS
_TOTAL = _FEATURES_SHAPE[0] * _FEATURES_SHAPE[1] * 9
_PAD_TOTAL = 22528
_APW = _PAD_TOTAL // _NW
_STEPS = _APW // _L
_BLK = 2
_GT_UNROLL = 2

_LN2 = 0.6931471805599453
_SQRT2 = 1.4142135623730951


def _generate_base_anchors(base_size=16):
    ratios = np.array([0.5, 1.0, 2.0])
    scales = np.array([8.0, 16.0, 32.0])
    w = float(base_size); h = float(base_size)
    x_ctr = 0.5 * (w - 1.0); y_ctr = 0.5 * (h - 1.0)
    size = w * h
    size_ratios = size / ratios
    ws = np.round(np.sqrt(size_ratios))
    hs = np.round(ws * ratios)
    ws = (ws[:, None] * scales[None, :]).reshape(-1)
    hs = (hs[:, None] * scales[None, :]).reshape(-1)
    anchors = np.stack([x_ctr - 0.5 * (ws - 1.0),
                        y_ctr - 0.5 * (hs - 1.0),
                        x_ctr + 0.5 * (ws - 1.0),
                        y_ctr + 0.5 * (hs - 1.0)], axis=1)
    return anchors.astype(np.float32)


def _shift_anchors(shape, stride, anchors):
    shift_x = np.arange(0, shape[1]) * stride
    shift_y = np.arange(0, shape[0]) * stride
    sx, sy = np.meshgrid(shift_x, shift_y)
    shifts = np.stack([sx.ravel(), sy.ravel(), sx.ravel(), sy.ravel()],
                      axis=1).astype(np.float32)
    return (anchors[None, :, :] + shifts[:, None, :]).reshape(-1, 4)


_ANCHORS_NP = _shift_anchors(_FEATURES_SHAPE, _STRIDE,
                             _generate_base_anchors(_ANCHOR_SIZE))


def _anchor_consts():
    a = np.concatenate(
        [_ANCHORS_NP, np.broadcast_to(_ANCHORS_NP[:1], (_PAD_TOTAL - _TOTAL, 4))],
        axis=0).astype(np.float64)
    x1, y1, x2, y2 = a[:, 0], a[:, 1], a[:, 2], a[:, 3]
    w = x2 - x1 + 1.0
    h = y2 - y1 + 1.0
    rows = np.stack([
        x1, y1, x2, y2,
        w * h,
        x1 + 0.5 * w,
        y1 + 0.5 * h,
        1.0 / w,
        1.0 / h,
        np.log(w),
        np.log(h),
        ((x1 >= 0.0) & (y1 >= 0.0)).astype(np.float64),
    ], axis=0).astype(np.float32)
    return np.ascontiguousarray(
        rows.reshape(12, _NW, _APW).transpose(1, 0, 2))


_ACONST_NP = _anchor_consts()

_AX1, _AY1, _AX2, _AY2, _AAREA, _ACX, _ACY, _AINVW, _AINVH, _ALOGW, _ALOGH, _AIB = range(12)


def _softlog(x):
    b = lax.bitcast_convert_type(x, jnp.int32)
    e = ((b >> 23) & 0xFF) - 127
    m = lax.bitcast_convert_type((b & 0x007FFFFF) | 0x3F800000, jnp.float32)
    big = m > _SQRT2
    m = jnp.where(big, m * 0.5, m)
    e = jnp.where(big, e + 1, e)
    s = (m - 1.0) / (m + 1.0)
    t = s * s
    p = 2.0 * s * (1.0 + t * (1.0 / 3.0 + t * (1.0 / 5.0 + t * (1.0 / 7.0))))
    return e.astype(jnp.float32) * _LN2 + p


def _sc_body(acst_hbm, graw_hbm, im_hbm,
             lab_hbm, bb_hbm, ct_hbm,
             acst_v, graw_v, im_v2,
             gx1_v, gy1_v, gx2_v, gy2_v, garea_v,
             gcx_v, gcy_v, glw_v, glh_v, gcls_v,
             lab_v, bb_v, ct_v):
    wid = lax.axis_index("s") * _NC + lax.axis_index("c")
    base = wid * _APW

    pltpu.sync_copy(acst_hbm.at[wid], acst_v)
    pltpu.sync_copy(graw_hbm, graw_v)
    pltpu.sync_copy(im_hbm, im_v2)

    zeros16 = jnp.zeros((_L,), jnp.float32)
    ones16 = jnp.ones((_L,), jnp.float32)
    lane = lax.iota(jnp.int32, _L)

    h_img = im_v2[0, :]
    w_img = im_v2[1, :]

    for t in range(_GT_PAD // _L):
        s = pl.ds(t * _L, _L)
        x1 = graw_v[0, s]; y1 = graw_v[1, s]
        x2 = graw_v[2, s]; y2 = graw_v[3, s]
        c = graw_v[4, s]
        gw = x2 - x1 + 1.0
        gh = y2 - y1 + 1.0
        gx1_v[s] = x1; gy1_v[s] = y1
        gx2_v[s] = x2; gy2_v[s] = y2
        garea_v[s] = gw * gh
        gcx_v[s] = x1 + 0.5 * gw
        gcy_v[s] = y1 + 0.5 * gh
        glw_v[s] = _softlog(gw)
        glh_v[s] = _softlog(gh)
        gcls_v[s] = c

    def blk_fn(blk, _):
        ax1 = []; ay1 = []; ax2 = []; ay2 = []; aarea = []
        for t in range(_BLK):
            s = pl.ds((blk * _BLK + t) * _L, _L)
            ax1.append(acst_v[_AX1, s]); ay1.append(acst_v[_AY1, s])
            ax2.append(acst_v[_AX2, s]); ay2.append(acst_v[_AY2, s])
            aarea.append(acst_v[_AAREA, s])

        def gt_fn(jj, carry):
            bi = list(carry[0:_BLK])
            bu = list(carry[_BLK:2 * _BLK])
            bx = list(carry[2 * _BLK:3 * _BLK])
            for u in range(_GT_UNROLL):
                j = jj * _GT_UNROLL + u
                jb = jnp.full((_L,), j, jnp.int32)
                gx1 = plsc.load_gather(gx1_v, [jb])
                gy1 = plsc.load_gather(gy1_v, [jb])
                gx2 = plsc.load_gather(gx2_v, [jb])
                gy2 = plsc.load_gather(gy2_v, [jb])
                garea = plsc.load_gather(garea_v, [jb])
                for t in range(_BLK):
                    iw = jnp.minimum(ax2[t], gx2) - jnp.maximum(ax1[t], gx1) + 1.0
                    ih = jnp.minimum(ay2[t], gy2) - jnp.maximum(ay1[t], gy1) + 1.0
                    iw = jnp.maximum(iw, 0.0)
                    ih = jnp.maximum(ih, 0.0)
                    inter = iw * ih
                    union = aarea[t] + garea - inter
                    m = inter * bu[t] > bi[t] * union
                    bi[t] = jnp.where(m, inter, bi[t])
                    bu[t] = jnp.where(m, union, bu[t])
                    bx[t] = jnp.where(m, jb, bx[t])
            return tuple(bi) + tuple(bu) + tuple(bx)

        init = (tuple(jnp.full((_L,), -1.0, jnp.float32) for _ in range(_BLK))
                + tuple(jnp.ones((_L,), jnp.float32) for _ in range(_BLK))
                + tuple(jnp.zeros((_L,), jnp.int32) for _ in range(_BLK)))
        carry = lax.fori_loop(0, _NUM_GT // _GT_UNROLL, gt_fn, init)

        for t in range(_BLK):
            step = blk * _BLK + t
            s = pl.ds(step * _L, _L)
            best = carry[t] / carry[_BLK + t]
            bidx = carry[2 * _BLK + t]

            gcx = plsc.load_gather(gcx_v, [bidx])
            gcy = plsc.load_gather(gcy_v, [bidx])
            glw = plsc.load_gather(glw_v, [bidx])
            glh = plsc.load_gather(glh_v, [bidx])
            gcls = plsc.load_gather(gcls_v, [bidx])

            lab = jnp.full((_L,), -2.0, jnp.float32)
            lab = jnp.where(best < _NEG_OVERLAP, -1.0, lab)
            lab = jnp.where(best >= _POS_OVERLAP, 1.0, lab)

            inside = ((acst_v[_AIB, s] > 0.0)
                      & (ax2[t] < w_img) & (ay2[t] < h_img))
            lab = jnp.where(inside, lab, -2.0)
            lab = jnp.where(lab == 1.0, gcls, lab)

            dx = (gcx - acst_v[_ACX, s]) * acst_v[_AINVW, s]
            dy = (gcy - acst_v[_ACY, s]) * acst_v[_AINVH, s]
            dw = glw - acst_v[_ALOGW, s]
            dh = glh - acst_v[_ALOGH, s]

            lab_v[s] = lab
            arow = step * _L + lane
            bb_v[pl.ds(0 * _APW + step * _L, _L)] = dx
            bb_v[pl.ds(1 * _APW + step * _L, _L)] = dy
            bb_v[pl.ds(2 * _APW + step * _L, _L)] = dw
            bb_v[pl.ds(3 * _APW + step * _L, _L)] = dh
            for k in range(_NUM_CLASSES):
                ct_v[pl.ds(k * _APW + step * _L, _L)] = zeros16
            cls_i = lab.astype(jnp.int32)
            valid = (cls_i >= 0) & (cls_i < _NUM_CLASSES)
            plsc.store_scatter(ct_v, [cls_i * _APW + arow], ones16,
                               mask=valid)
        return 0

    lax.fori_loop(0, _STEPS // _BLK, blk_fn, 0)

    pltpu.sync_copy(lab_v, lab_hbm.at[pl.ds(base, _APW)])
    for c in range(4):
        pltpu.sync_copy(bb_v.at[pl.ds(c * _APW, _APW)],
                        bb_hbm.at[pl.ds(c * _PAD_TOTAL + base, _APW)])
    for c in range(_NUM_CLASSES):
        pltpu.sync_copy(ct_v.at[pl.ds(c * _APW, _APW)],
                        ct_hbm.at[pl.ds(c * _PAD_TOTAL + base, _APW)])


@functools.partial(jax.jit, static_argnames=())
def _run_sc(acst, graw, im_pad):
    mesh = plsc.VectorSubcoreMesh(core_axis_name="c", subcore_axis_name="s",
                                  num_cores=_NC, num_subcores=_NS)
    f = pl.kernel(
        _sc_body,
        out_type=(
            jax.ShapeDtypeStruct((_PAD_TOTAL,), jnp.float32),
            jax.ShapeDtypeStruct((_PAD_TOTAL * 4,), jnp.float32),
            jax.ShapeDtypeStruct((_PAD_TOTAL * _NUM_CLASSES,), jnp.float32),
        ),
        mesh=mesh,
        compiler_params=pltpu.CompilerParams(needs_layout_passes=False),
        scratch_types=[
            pltpu.VMEM((12, _APW), jnp.float32),
            pltpu.VMEM((5, _GT_PAD), jnp.float32),
            pltpu.VMEM((2, _L), jnp.float32),
            pltpu.VMEM((_GT_PAD,), jnp.float32),
            pltpu.VMEM((_GT_PAD,), jnp.float32),
            pltpu.VMEM((_GT_PAD,), jnp.float32),
            pltpu.VMEM((_GT_PAD,), jnp.float32),
            pltpu.VMEM((_GT_PAD,), jnp.float32),
            pltpu.VMEM((_GT_PAD,), jnp.float32),
            pltpu.VMEM((_GT_PAD,), jnp.float32),
            pltpu.VMEM((_GT_PAD,), jnp.float32),
            pltpu.VMEM((_GT_PAD,), jnp.float32),
            pltpu.VMEM((_GT_PAD,), jnp.float32),
            pltpu.VMEM((_APW,), jnp.float32),
            pltpu.VMEM((_APW * 4,), jnp.float32),
            pltpu.VMEM((_APW * _NUM_CLASSES,), jnp.float32),
        ],
    )
    return f(acst, graw, im_pad)


def kernel(im_info, gt_boxes):
    gt = gt_boxes[0]
    graw = jnp.concatenate(
        [gt.T, jnp.zeros((5, _GT_PAD - _NUM_GT), jnp.float32)], axis=1)
    im_pad = jnp.broadcast_to(im_info[0, :2, None], (2, _L))
    acst = jnp.asarray(_ACONST_NP)

    lab, bb, ct = _run_sc(acst, graw, im_pad)

    labels = lab[:_TOTAL][None]
    class_targets = ct.reshape(_NUM_CLASSES, _PAD_TOTAL)[:, :_TOTAL].T[None]
    bbox_targets = bb.reshape(4, _PAD_TOTAL)[:, :_TOTAL].T[None]
    anchors = jnp.asarray(_ANCHORS_NP)[None]
    return (labels, class_targets, bbox_targets, anchors)
